# Initial kernel scaffold; baseline (speedup 1.0000x reference)
import jax
import jax.numpy as jnp
from jax.experimental import pallas as pl


def kernel(x, edge_index, edge_attr, edge_label_index, edge_label_attr, W1l, b1, W1r, W2l, b2, W2r, Wm1, bm1, Wm2, bm2):
    raise NotImplementedError("write your pallas kernel here")


# speedup vs baseline: 2.9790x; 2.9790x over previous
import functools

import jax
import jax.numpy as jnp
from jax import lax
from jax.experimental import pallas as pl
from jax.experimental.pallas import tpu as pltpu
from jax.experimental.pallas import tpu_sc as plsc

N = 10000
NP = 10240
E = 320000
EPR = 2528
L = 100000
LPR = 800
LP = LPR * 128
W1 = 144
NTILE = 16
ROWS_PER_TILE_OUT = NP // NTILE


def _agg1(xaug_hbm, srcm_hbm, dstm_hbm, zeros_hbm, out_hbm,
          src_v, dst_v, rows_v, acc_sh, sem):
    c = lax.axis_index("c")
    s = lax.axis_index("s")
    pltpu.sync_copy(zeros_hbm, acc_sh.at[pl.ds(s * ROWS_PER_TILE_OUT, ROWS_PER_TILE_OUT)])
    plsc.subcore_barrier()
    w = s * 2 + c
    base = w * (EPR // 32)

    def body(i, carry):
        r = base + i
        pltpu.sync_copy(srcm_hbm.at[r], src_v)
        pltpu.sync_copy(dstm_hbm.at[r], dst_v)
        pltpu.async_copy(xaug_hbm.at[src_v], rows_v, sem).wait()
        pltpu.sync_copy(rows_v, acc_sh.at[dst_v], add=True)
        return carry

    lax.fori_loop(0, EPR // 32, body, 0)
    plsc.subcore_barrier()
    pltpu.sync_copy(acc_sh.at[pl.ds(s * ROWS_PER_TILE_OUT, ROWS_PER_TILE_OUT)],
                    out_hbm.at[c].at[pl.ds(s * ROWS_PER_TILE_OUT, ROWS_PER_TILE_OUT)])


def _agg2(h1cat_hbm, srcm2_hbm, dstm_hbm, zeros_hbm, out_hbm,
          src_v, dst_v, rows_v, acc_sh, sem):
    c = lax.axis_index("c")
    s = lax.axis_index("s")
    pltpu.sync_copy(zeros_hbm, acc_sh.at[pl.ds(s * ROWS_PER_TILE_OUT, ROWS_PER_TILE_OUT)])
    plsc.subcore_barrier()
    base = s * (EPR // 16)

    def body(i, carry):
        r = base + i
        pltpu.sync_copy(srcm2_hbm.at[c].at[r], src_v)
        pltpu.sync_copy(dstm_hbm.at[r], dst_v)
        pltpu.async_copy(h1cat_hbm.at[src_v], rows_v, sem).wait()
        pltpu.sync_copy(rows_v, acc_sh.at[dst_v], add=True)
        return carry

    lax.fori_loop(0, EPR // 16, body, 0)
    plsc.subcore_barrier()
    pltpu.sync_copy(acc_sh.at[pl.ds(s * ROWS_PER_TILE_OUT, ROWS_PER_TILE_OUT)],
                    out_hbm.at[c].at[pl.ds(s * ROWS_PER_TILE_OUT, ROWS_PER_TILE_OUT)])


def _gather_labels(pcat_hbm, lidx_hbm, out_hbm, idx_v, buf_v, sem):
    c = lax.axis_index("c")
    s = lax.axis_index("s")
    base = s * (LPR // 16)

    def body(i, carry):
        r = base + i
        pltpu.sync_copy(lidx_hbm.at[c].at[r], idx_v)
        pltpu.async_copy(pcat_hbm.at[idx_v], buf_v, sem).wait()
        pltpu.sync_copy(buf_v, out_hbm.at[c].at[pl.ds(r * 128, 128)])
        return carry

    lax.fori_loop(0, LPR // 16, body, 0)


@functools.lru_cache(maxsize=None)
def _sc_calls():
    mesh = plsc.VectorSubcoreMesh(core_axis_name="c", subcore_axis_name="s")
    cp = pltpu.CompilerParams(use_tc_tiling_on_sc=False)
    agg1_call = functools.partial(
        pl.kernel, mesh=mesh,
        out_type=jax.ShapeDtypeStruct((2, NP, W1), jnp.float32),
        scratch_types=[
            pltpu.VMEM((128,), jnp.int32),
            pltpu.VMEM((128,), jnp.int32),
            pltpu.VMEM((128, W1), jnp.float32),
            pltpu.VMEM_SHARED((NP, W1), jnp.float32),
            pltpu.SemaphoreType.DMA,
        ],
        compiler_params=cp,
    )(_agg1)
    agg2_call = functools.partial(
        pl.kernel, mesh=mesh,
        out_type=jax.ShapeDtypeStruct((2, NP, 128), jnp.float32),
        scratch_types=[
            pltpu.VMEM((128,), jnp.int32),
            pltpu.VMEM((128,), jnp.int32),
            pltpu.VMEM((128, 128), jnp.float32),
            pltpu.VMEM_SHARED((NP, 128), jnp.float32),
            pltpu.SemaphoreType.DMA,
        ],
        compiler_params=cp,
    )(_agg2)
    gather_call = functools.partial(
        pl.kernel, mesh=mesh,
        out_type=jax.ShapeDtypeStruct((2, LP, 32), jnp.float32),
        scratch_types=[
            pltpu.VMEM((128,), jnp.int32),
            pltpu.VMEM((128, 32), jnp.float32),
            pltpu.SemaphoreType.DMA,
        ],
        compiler_params=cp,
    )(_gather_labels)
    return agg1_call, agg2_call, gather_call


BN = 256
NPB = NP // BN


def _mm1_body(agg_ref, xaug_ref, w1l_ref, w1r_ref, b1_ref, h1_ref, invc_ref):
    a = agg_ref[0] + agg_ref[1]
    cnt = a[:, 128:129]
    inv = 1.0 / jnp.maximum(cnt, 1.0)
    mean = a[:, :128] * inv
    x = xaug_ref[:, :128]
    h = mean @ w1l_ref[...] + x @ w1r_ref[...] + b1_ref[0]
    h1_ref[...] = jnp.maximum(h, 0.0)
    invc_ref[...] = jnp.broadcast_to(inv, (BN, 128))


def _mm2_body(agg_ref, invc_ref, h1a_ref, h1b_ref, w2l_ref, w2r_ref, b2_ref,
              wm1_ref, pcat_ref):
    inv = invc_ref[:, :1]
    ma = agg_ref[0] * inv
    mb = agg_ref[1] * inv
    pre = (ma @ w2l_ref[:128] + mb @ w2l_ref[128:]
           + h1a_ref[...] @ w2r_ref[:128] + h1b_ref[...] @ w2r_ref[128:]
           + b2_ref[...])
    h2 = jnp.maximum(pre, 0.0)
    pcat_ref[...] = h2 @ wm1_ref[0]


BL = 2048
LPB = LP // BL


def _mm3_body(g_ref, attr_ref, wattr_ref, bm1_ref, wm2_ref, bm2_ref, out_ref):
    hidden = jnp.maximum(g_ref[0] + g_ref[1] + attr_ref[...] @ wattr_ref[...]
                         + bm1_ref[...], 0.0)
    out_ref[...] = hidden @ wm2_ref[...] + bm2_ref[...]


def kernel(x, edge_index, edge_attr, edge_label_index, edge_label_attr,
           W1l, b1, W1r, W2l, b2, W2r, Wm1, bm1, Wm2, bm2):
    f32 = jnp.float32
    src = edge_index[0]
    dst = edge_index[1]

    xaug = jnp.concatenate(
        [x, jnp.ones((N, 1), f32), jnp.zeros((N, W1 - 129), f32)], axis=1)
    xaug = jnp.concatenate([xaug, jnp.zeros((NP - N, W1), f32)], axis=0)

    EP = EPR * 128
    srcm = jnp.concatenate([src, jnp.zeros((EP - E,), jnp.int32)]).reshape(EPR, 128)
    dstm = jnp.concatenate([dst, jnp.full((EP - E,), N, jnp.int32)]).reshape(EPR, 128)
    srcm2 = jnp.stack([srcm, srcm + NP])

    sl = edge_label_index[:, 0]
    dl = edge_label_index[:, 1]
    slp = jnp.concatenate([sl, jnp.zeros((LP - L,), jnp.int32)]).reshape(LPR, 128)
    dlp = jnp.concatenate([dl, jnp.zeros((LP - L,), jnp.int32)]).reshape(LPR, 128)
    lidx = jnp.stack([slp, dlp + NP])

    attrp = jnp.concatenate(
        [edge_label_attr, jnp.zeros((LP - L, 16), f32)], axis=0)

    z1 = jnp.zeros((ROWS_PER_TILE_OUT, W1), f32)
    z2 = jnp.zeros((ROWS_PER_TILE_OUT, 128), f32)

    b1m = b1.reshape(2, 1, 128)
    b2m = b2.reshape(1, 256)
    wm1sd = jnp.stack([Wm1[:256], Wm1[256:512]])
    wattr = Wm1[512:]
    bm1m = bm1.reshape(1, 32)
    bm2m = bm2.reshape(1, 2)

    agg1_call, agg2_call, gather_call = _sc_calls()

    agg1 = agg1_call(xaug, srcm, dstm, z1)

    h1cat, invc = pl.pallas_call(
        _mm1_body,
        grid=(2, NPB),
        in_specs=[
            pl.BlockSpec((2, BN, W1), lambda j, i: (0, i, 0)),
            pl.BlockSpec((BN, W1), lambda j, i: (i, 0)),
            pl.BlockSpec((128, 128), lambda j, i: (0, j)),
            pl.BlockSpec((128, 128), lambda j, i: (0, j)),
            pl.BlockSpec((1, 1, 128), lambda j, i: (j, 0, 0)),
        ],
        out_specs=[
            pl.BlockSpec((BN, 128), lambda j, i: (j * NPB + i, 0)),
            pl.BlockSpec((BN, 128), lambda j, i: (i, 0)),
        ],
        out_shape=[
            jax.ShapeDtypeStruct((2 * NP, 128), f32),
            jax.ShapeDtypeStruct((NP, 128), f32),
        ],
    )(agg1, xaug, W1l, W1r, b1m)

    agg2 = agg2_call(h1cat, srcm2, dstm, z2)

    pcat = pl.pallas_call(
        _mm2_body,
        grid=(2, NPB),
        in_specs=[
            pl.BlockSpec((2, BN, 128), lambda j, i: (0, i, 0)),
            pl.BlockSpec((BN, 128), lambda j, i: (i, 0)),
            pl.BlockSpec((BN, 128), lambda j, i: (i, 0)),
            pl.BlockSpec((BN, 128), lambda j, i: (NPB + i, 0)),
            pl.BlockSpec((256, 256), lambda j, i: (0, 0)),
            pl.BlockSpec((256, 256), lambda j, i: (0, 0)),
            pl.BlockSpec((1, 256), lambda j, i: (0, 0)),
            pl.BlockSpec((1, 256, 32), lambda j, i: (j, 0, 0)),
        ],
        out_specs=pl.BlockSpec((BN, 32), lambda j, i: (j * NPB + i, 0)),
        out_shape=jax.ShapeDtypeStruct((2 * NP, 32), f32),
    )(agg2, invc, h1cat, h1cat, W2l, W2r, b2m, wm1sd)

    g = gather_call(pcat, lidx)

    outp = pl.pallas_call(
        _mm3_body,
        grid=(LPB,),
        in_specs=[
            pl.BlockSpec((2, BL, 32), lambda i: (0, i, 0)),
            pl.BlockSpec((BL, 16), lambda i: (i, 0)),
            pl.BlockSpec((16, 32), lambda i: (0, 0)),
            pl.BlockSpec((1, 32), lambda i: (0, 0)),
            pl.BlockSpec((32, 2), lambda i: (0, 0)),
            pl.BlockSpec((1, 2), lambda i: (0, 0)),
        ],
        out_specs=pl.BlockSpec((BL, 2), lambda i: (i, 0)),
        out_shape=jax.ShapeDtypeStruct((LP, 2), f32),
    )(g, attrp, wattr, bm1m, Wm2, bm2m)

    return outp[:L]


# speedup vs baseline: 3.2158x; 1.0795x over previous
import functools

import jax
import jax.numpy as jnp
from jax import lax
from jax.experimental import pallas as pl
from jax.experimental.pallas import tpu as pltpu
from jax.experimental.pallas import tpu_sc as plsc

N = 10000
NP = 10240
E = 320000
EPR = 2560
L = 100000
LPR = 832
LP = LPR * 128
NTILE = 16
RPT_OUT = NP // NTILE
NBUF = 4
GROWS = LPR // 16


def _make_agg(width, chunk):
    nrows_total = (EPR * 128) // chunk

    def body(table_hbm, srcm_hbm, dstm_hbm, zeros_hbm, out_hbm,
             s0, s1, s2, s3, d0, d1, d2, d3, b0, b1, b2, b3, acc_sh,
             g0, g1, g2, g3, i0, i1, i2, i3):
        c = lax.axis_index("c")
        s = lax.axis_index("s")
        sis = (s0, s1, s2, s3)
        dis = (d0, d1, d2, d3)
        bufs = (b0, b1, b2, b3)
        gsems = (g0, g1, g2, g3)
        isems = (i0, i1, i2, i3)
        pltpu.sync_copy(zeros_hbm, acc_sh.at[pl.ds(s * RPT_OUT, RPT_OUT)])
        base = s * (nrows_total // NTILE)

        def fire_idx(i, b):
            pltpu.async_copy(srcm_hbm.at[c].at[base + i], sis[b], isems[b])
            pltpu.async_copy(dstm_hbm.at[base + i], dis[b], isems[b])

        plsc.subcore_barrier()
        for b in range(NBUF):
            fire_idx(b, b)

        ngroups = (nrows_total // NTILE) // NBUF

        def loop(g, carry):
            base_i = g * NBUF
            for b in range(NBUF):
                pltpu.make_async_copy(srcm_hbm.at[c].at[base + base_i + b],
                                      sis[b], isems[b]).wait()
                pltpu.make_async_copy(dstm_hbm.at[base + base_i + b],
                                      dis[b], isems[b]).wait()
                pltpu.async_copy(table_hbm.at[sis[b]], bufs[b], gsems[b])
            for b in range(NBUF):
                pltpu.make_async_copy(table_hbm.at[sis[b]], bufs[b],
                                      gsems[b]).wait()
                pltpu.sync_copy(bufs[b], acc_sh.at[dis[b]], add=True)

                @pl.when(g + 1 < ngroups)
                def _():
                    fire_idx(base_i + NBUF + b, b)
            return carry

        lax.fori_loop(0, ngroups, loop, 0)
        plsc.subcore_barrier()
        pltpu.sync_copy(acc_sh.at[pl.ds(s * RPT_OUT, RPT_OUT)],
                        out_hbm.at[c].at[pl.ds(s * RPT_OUT, RPT_OUT)])

    mesh = plsc.VectorSubcoreMesh(core_axis_name="c", subcore_axis_name="s")
    return functools.partial(
        pl.kernel, mesh=mesh,
        out_type=jax.ShapeDtypeStruct((2, NP, width), jnp.float32),
        scratch_types=(
            [pltpu.VMEM((chunk,), jnp.int32)] * (2 * NBUF)
            + [pltpu.VMEM((chunk, width), jnp.float32)] * NBUF
            + [pltpu.VMEM_SHARED((NP, width), jnp.float32)]
            + [pltpu.SemaphoreType.DMA] * (2 * NBUF)
        ),
        compiler_params=pltpu.CompilerParams(use_tc_tiling_on_sc=False),
    )(body)


def _gather_labels(pcat_hbm, lidx_hbm, out_hbm,
                   idx_v, b0, b1, b2, b3, g0, g1, g2, g3, s0, s1, s2, s3):
    c = lax.axis_index("c")
    s = lax.axis_index("s")
    base = s * GROWS
    pltpu.sync_copy(lidx_hbm.at[c].at[pl.ds(base, GROWS)], idx_v)
    bufs = (b0, b1, b2, b3)
    gsems = (g0, g1, g2, g3)
    ssems = (s0, s1, s2, s3)

    def fire_gather(i, b):
        pltpu.async_copy(pcat_hbm.at[idx_v.at[i]], bufs[b], gsems[b])

    for b in range(NBUF):
        fire_gather(b, b)

    ngroups = GROWS // NBUF

    def body(g, carry):
        i0 = g * NBUF
        for b in range(NBUF):
            pltpu.make_async_copy(pcat_hbm.at[idx_v.at[i0 + b]],
                                  bufs[b], gsems[b]).wait()
            pltpu.async_copy(
                bufs[b], out_hbm.at[c].at[pl.ds((base + i0 + b) * 128, 128)],
                ssems[b])
        for b in range(NBUF):
            pltpu.make_async_copy(
                bufs[b], out_hbm.at[c].at[pl.ds((base + i0 + b) * 128, 128)],
                ssems[b]).wait()

            @pl.when(g + 1 < ngroups)
            def _():
                fire_gather(i0 + NBUF + b, b)
        return carry

    lax.fori_loop(0, ngroups, body, 0)


@functools.lru_cache(maxsize=None)
def _sc_calls():
    agg1_call = _make_agg(80, 128)
    agg2_call = _make_agg(128, 64)
    mesh = plsc.VectorSubcoreMesh(core_axis_name="c", subcore_axis_name="s")
    gather_call = functools.partial(
        pl.kernel, mesh=mesh,
        out_type=jax.ShapeDtypeStruct((2, LP, 32), jnp.float32),
        scratch_types=(
            [pltpu.VMEM((GROWS, 128), jnp.int32)]
            + [pltpu.VMEM((128, 32), jnp.float32)] * NBUF
            + [pltpu.SemaphoreType.DMA] * (2 * NBUF)
        ),
        compiler_params=pltpu.CompilerParams(use_tc_tiling_on_sc=False),
    )(_gather_labels)
    return agg1_call, agg2_call, gather_call


BN = 1024
NPB = NP // BN


def _mm1_body(agg_ref, x_ref, w1l_ref, w1r_ref, b1_ref, h1_ref, invc_ref):
    s_lo = agg_ref[0]
    s_hi = agg_ref[1][:, :48]
    cnt = agg_ref[1][:, 48:49]
    inv = 1.0 / jnp.maximum(cnt, 1.0)
    mean = jnp.concatenate([s_lo, s_hi], axis=1) * inv
    h = mean @ w1l_ref[...] + x_ref[...] @ w1r_ref[...] + b1_ref[0]
    h1_ref[...] = jnp.maximum(h, 0.0)
    invc_ref[...] = jnp.broadcast_to(inv, (BN, 128))


def _mm2_body(agg_ref, invc_ref, h1a_ref, h1b_ref, w2l_ref, w2r_ref, b2_ref,
              wm1_ref, pcat_ref):
    inv = invc_ref[:, :1]
    ma = agg_ref[0] * inv
    mb = agg_ref[1] * inv
    pre = (ma @ w2l_ref[:128] + mb @ w2l_ref[128:]
           + h1a_ref[...] @ w2r_ref[:128] + h1b_ref[...] @ w2r_ref[128:]
           + b2_ref[...])
    h2 = jnp.maximum(pre, 0.0)
    pcat_ref[...] = h2 @ wm1_ref[0]


BL = 4096
LPB = LP // BL


def _mm3_body(g_ref, attr_ref, wattr_ref, bm1_ref, wm2_ref, bm2_ref, out_ref):
    hidden = jnp.maximum(g_ref[0] + g_ref[1] + attr_ref[...] @ wattr_ref[...]
                         + bm1_ref[...], 0.0)
    out_ref[...] = hidden @ wm2_ref[...] + bm2_ref[...]


def kernel(x, edge_index, edge_attr, edge_label_index, edge_label_attr,
           W1l, b1, W1r, W2l, b2, W2r, Wm1, bm1, Wm2, bm2):
    f32 = jnp.float32
    src = edge_index[0]
    dst = edge_index[1]

    rowpad = jnp.zeros((NP - N, 80), f32)
    xa = jnp.concatenate([x[:, :80], rowpad], axis=0)
    xb = jnp.concatenate([
        jnp.concatenate([x[:, 80:], jnp.ones((N, 1), f32),
                         jnp.zeros((N, 31), f32)], axis=1),
        rowpad], axis=0)
    xcat = jnp.concatenate([xa, xb], axis=0)
    xp = jnp.concatenate([x, jnp.zeros((NP - N, 128), f32)], axis=0)

    EP = EPR * 128
    srcp = jnp.concatenate([src, jnp.zeros((EP - E,), jnp.int32)])
    dstp = jnp.concatenate([dst, jnp.full((EP - E,), N, jnp.int32)])
    srcm2 = jnp.stack([srcp, srcp + NP]).reshape(2, EPR, 128)
    dstm = dstp.reshape(EPR, 128)
    srcm2_64 = srcm2.reshape(2, EPR * 2, 64)
    dstm_64 = dstp.reshape(EPR * 2, 64)

    sl = edge_label_index[:, 0]
    dl = edge_label_index[:, 1]
    slp = jnp.concatenate([sl, jnp.zeros((LP - L,), jnp.int32)]).reshape(LPR, 128)
    dlp = jnp.concatenate([dl, jnp.zeros((LP - L,), jnp.int32)]).reshape(LPR, 128)
    lidx = jnp.stack([slp, dlp + NP])

    attrp = jnp.concatenate(
        [edge_label_attr, jnp.zeros((LP - L, 16), f32)], axis=0)

    z80 = jnp.zeros((RPT_OUT, 80), f32)
    z128 = jnp.zeros((RPT_OUT, 128), f32)

    b1m = b1.reshape(2, 1, 128)
    b2m = b2.reshape(1, 256)
    wm1sd = jnp.stack([Wm1[:256], Wm1[256:512]])
    wattr = Wm1[512:]
    bm1m = bm1.reshape(1, 32)
    bm2m = bm2.reshape(1, 2)

    agg1_call, agg2_call, gather_call = _sc_calls()

    agg1 = agg1_call(xcat, srcm2, dstm, z80)

    h1cat, invc = pl.pallas_call(
        _mm1_body,
        grid=(2, NPB),
        in_specs=[
            pl.BlockSpec((2, BN, 80), lambda j, i: (0, i, 0)),
            pl.BlockSpec((BN, 128), lambda j, i: (i, 0)),
            pl.BlockSpec((128, 128), lambda j, i: (0, j)),
            pl.BlockSpec((128, 128), lambda j, i: (0, j)),
            pl.BlockSpec((1, 1, 128), lambda j, i: (j, 0, 0)),
        ],
        out_specs=[
            pl.BlockSpec((BN, 128), lambda j, i: (j * NPB + i, 0)),
            pl.BlockSpec((BN, 128), lambda j, i: (i, 0)),
        ],
        out_shape=[
            jax.ShapeDtypeStruct((2 * NP, 128), f32),
            jax.ShapeDtypeStruct((NP, 128), f32),
        ],
    )(agg1, xp, W1l, W1r, b1m)

    agg2 = agg2_call(h1cat, srcm2_64, dstm_64, z128)

    pcat = pl.pallas_call(
        _mm2_body,
        grid=(2, NPB),
        in_specs=[
            pl.BlockSpec((2, BN, 128), lambda j, i: (0, i, 0)),
            pl.BlockSpec((BN, 128), lambda j, i: (i, 0)),
            pl.BlockSpec((BN, 128), lambda j, i: (i, 0)),
            pl.BlockSpec((BN, 128), lambda j, i: (NPB + i, 0)),
            pl.BlockSpec((256, 256), lambda j, i: (0, 0)),
            pl.BlockSpec((256, 256), lambda j, i: (0, 0)),
            pl.BlockSpec((1, 256), lambda j, i: (0, 0)),
            pl.BlockSpec((1, 256, 32), lambda j, i: (j, 0, 0)),
        ],
        out_specs=pl.BlockSpec((BN, 32), lambda j, i: (j * NPB + i, 0)),
        out_shape=jax.ShapeDtypeStruct((2 * NP, 32), f32),
    )(agg2, invc, h1cat, h1cat, W2l, W2r, b2m, wm1sd)

    g = gather_call(pcat, lidx)

    outp = pl.pallas_call(
        _mm3_body,
        grid=(LPB,),
        in_specs=[
            pl.BlockSpec((2, BL, 32), lambda i: (0, i, 0)),
            pl.BlockSpec((BL, 16), lambda i: (i, 0)),
            pl.BlockSpec((16, 32), lambda i: (0, 0)),
            pl.BlockSpec((1, 32), lambda i: (0, 0)),
            pl.BlockSpec((32, 2), lambda i: (0, 0)),
            pl.BlockSpec((1, 2), lambda i: (0, 0)),
        ],
        out_specs=pl.BlockSpec((BL, 2), lambda i: (i, 0)),
        out_shape=jax.ShapeDtypeStruct((LP, 2), f32),
    )(g, attrp, wattr, bm1m, Wm2, bm2m)

    return outp[:L]
